# baseline (speedup 1.0000x reference)
import functools

import jax
import jax.numpy as jnp
from jax import lax
from jax.experimental import pallas as pl
from jax.experimental.pallas import tpu as pltpu
from jax.experimental.pallas import tpu_sc as plsc

N = 10000
D = 256
E = 160000
NC = 2
NS = 16
L = 16
NH = 5120
NP = NC * NH
TR = NH // NS
EC = E // NS
G = 128
C = 2000
PCAP = C + G + 48
NB = PCAP // G
QH = NH // 2
TQ = QH // NS
XB = 64

_MESH = plsc.VectorSubcoreMesh(
    core_axis_name="c", subcore_axis_name="s", num_cores=NC, num_subcores=NS)

_MAGIC = 0x5F3759DF


def _rsqrt_newton(v):
    i = plsc.bitcast(v, jnp.int32)
    i = jnp.int32(_MAGIC) - lax.shift_right_logical(i, 1)
    y = plsc.bitcast(i, jnp.float32)
    for _ in range(3):
        y = y * (1.5 - 0.5 * v * y * y)
    return y


def _k1_body(dst_hbm, x_hbm, dinv_hbm, y_hbm,
             dst_v, hist_v, part_v, red_v, dinv_v, xbuf, hist_sh):
    c = lax.axis_index("c")
    s = lax.axis_index("s")
    lo = c * NH
    lanes = lax.iota(jnp.int32, 16)
    ones = jnp.full((L,), 1.0, jnp.float32)

    def zero_body(j, _):
        for l in range(16):
            hist_v[pl.ds(l * NH + j * L, L)] = jnp.zeros((L,), jnp.float32)
        return 0
    lax.fori_loop(0, NH // L, zero_body, 0)

    pltpu.sync_copy(dst_hbm.at[pl.ds(s * EC, EC)], dst_v)
    lane_base = lanes * NH

    def hist_body(j, _):
        d = dst_v[pl.ds(j * L, L)]
        m = (d >= lo) & (d < lo + NH)
        dloc = jnp.where(m, d - lo, 0)
        plsc.addupdate_scatter(hist_v, [lane_base + dloc], ones, mask=m)
        return 0
    lax.fori_loop(0, EC // L, hist_body, 0)

    def lred_body(j, _):
        acc = hist_v[pl.ds(j * L, L)]
        for l in range(1, 16):
            acc = acc + hist_v[pl.ds(l * NH + j * L, L)]
        part_v[pl.ds(j * L, L)] = acc
        return 0
    lax.fori_loop(0, NH // L, lred_body, 0)

    pltpu.sync_copy(part_v, hist_sh.at[pl.ds(s * NH, NH)])
    plsc.subcore_barrier()
    for t in range(NS):
        pltpu.sync_copy(hist_sh.at[pl.ds(t * NH + s * TR, TR)], red_v.at[t])

    def deg_body(j, _):
        acc = red_v[0, pl.ds(j * L, L)]
        for t in range(1, 16):
            acc = acc + red_v[t, pl.ds(j * L, L)]
        deg = acc + 1.0
        dinv_v[pl.ds(j * L, L)] = _rsqrt_newton(deg)
        return 0
    lax.fori_loop(0, TR // L, deg_body, 0)

    g0 = lo + s * TR
    pltpu.sync_copy(dinv_v, dinv_hbm.at[pl.ds(g0, TR)])

    def y_body(jb, _):
        base = jb * XB
        pltpu.sync_copy(x_hbm.at[pl.ds(g0 + base, XB)], xbuf)
        for q in range(XB // L):
            ch = dinv_v[pl.ds(base + q * L, L)]
            for k in range(L):
                sv = jnp.full((L,), ch[k], jnp.float32)
                r = q * L + k
                for v in range(D // L):
                    xbuf[r, pl.ds(v * L, L)] = xbuf[r, pl.ds(v * L, L)] * sv
        pltpu.sync_copy(xbuf, y_hbm.at[pl.ds(g0 + base, XB)])
        return 0
    lax.fori_loop(0, TR // XB, y_body, 0)


@functools.partial(
    pl.kernel,
    out_type=(jax.ShapeDtypeStruct((NP,), jnp.float32),
              jax.ShapeDtypeStruct((NP, D), jnp.float32)),
    mesh=_MESH,
    compiler_params=pltpu.CompilerParams(needs_layout_passes=False, use_tc_tiling_on_sc=False),
    scratch_types=[
        pltpu.VMEM((EC,), jnp.int32),
        pltpu.VMEM((16 * NH,), jnp.float32),
        pltpu.VMEM((NH,), jnp.float32),
        pltpu.VMEM((NS, TR), jnp.float32),
        pltpu.VMEM((TR,), jnp.float32),
        pltpu.VMEM((XB, D), jnp.float32),
        pltpu.VMEM_SHARED((NS * NH,), jnp.float32),
    ],
)
def _k1(dst_hbm, x_hbm, dinv_hbm, y_hbm, *scratch):
    _k1_body(dst_hbm, x_hbm, dinv_hbm, y_hbm, *scratch)


def _k2_body(src_hbm, dst_hbm, y_hbm, s_hbm,
             src_v, dst_v, psrc, pdst1, pdst2, gbuf0, gbuf1, sem0, sem1,
             acc_sh):
    cc = lax.axis_index("c")
    s = lax.axis_index("s")

    dumm_s = jnp.full((L,), NP - L, jnp.int32)
    dumm_d = jnp.zeros((L,), jnp.int32)

    def gstart(b, buf, sem):
        pltpu.async_copy(y_hbm.at[psrc.at[pl.ds(b * G, G)]], buf, sem)

    def gwait(b, buf, sem):
        pltpu.make_async_copy(
            y_hbm.at[psrc.at[pl.ds(b * G, G)]], buf, sem).wait()

    for p in range(2):
        lo = (cc * 2 + p) * QH

        def zg_body(j, _):
            for v in range(D // L):
                gbuf0[j, pl.ds(v * L, L)] = jnp.zeros((L,), jnp.float32)
            return 0
        lax.fori_loop(0, G, zg_body, 0)
        pltpu.sync_copy(gbuf0, acc_sh.at[pl.ds(s * TQ, G)])
        pltpu.sync_copy(gbuf0.at[pl.ds(0, TQ - G)],
                        acc_sh.at[pl.ds(s * TQ + G, TQ - G)])
        plsc.subcore_barrier()

        def sub_body(k, _):
            e0 = s * EC + k * C
            pltpu.sync_copy(src_hbm.at[pl.ds(e0, C)], src_v)
            pltpu.sync_copy(dst_hbm.at[pl.ds(e0, C)], dst_v)

            def scan_body(j, cnt):
                d = dst_v[pl.ds(j * L, L)]
                sv = src_v[pl.ds(j * L, L)]
                m = (d >= lo) & (d < lo + QH)
                plsc.store_compressed(psrc.at[pl.ds(cnt, L)], sv, mask=m)
                plsc.store_compressed(pdst1.at[pl.ds(cnt, L)], d - lo, mask=m)
                return cnt + plsc.all_reduce_population_count(m)[0]
            cnt = lax.fori_loop(0, C // L, scan_body, jnp.int32(0))

            for j in range(G // L):
                psrc[pl.ds(cnt + j * L, L)] = dumm_s
                pdst1[pl.ds(cnt + j * L, L)] = dumm_d

            nb = (cnt + (G - 1)) // G

            def p2_body(b, _):
                for t in range(G // L):
                    pdst2[b, pl.ds(t * L, L)] = pdst1[pl.ds(b * G + t * L, L)]
                return 0
            lax.fori_loop(0, nb, p2_body, 0)

            def drain_body(b, _):
                pltpu.sync_copy(y_hbm.at[psrc.at[pl.ds(b * G, G)]], gbuf0)
                pltpu.sync_copy(gbuf0, acc_sh.at[pdst2.at[b]], add=True)
                return 0
            lax.fori_loop(0, nb, drain_body, 0)
            return 0
        lax.fori_loop(0, EC // C, sub_body, 0)

        plsc.subcore_barrier()

        g0 = lo + s * TQ
        pltpu.sync_copy(acc_sh.at[pl.ds(s * TQ, G)], gbuf0)
        pltpu.sync_copy(gbuf0, s_hbm.at[pl.ds(g0, G)])
        pltpu.sync_copy(acc_sh.at[pl.ds(s * TQ + G, TQ - G)],
                        gbuf1.at[pl.ds(0, TQ - G)])
        pltpu.sync_copy(gbuf1.at[pl.ds(0, TQ - G)],
                        s_hbm.at[pl.ds(g0 + G, TQ - G)])


@functools.partial(
    pl.kernel,
    out_type=jax.ShapeDtypeStruct((NP, D), jnp.float32),
    mesh=_MESH,
    compiler_params=pltpu.CompilerParams(needs_layout_passes=False, use_tc_tiling_on_sc=False),
    scratch_types=[
        pltpu.VMEM((C,), jnp.int32),
        pltpu.VMEM((C,), jnp.int32),
        pltpu.VMEM((PCAP,), jnp.int32),
        pltpu.VMEM((PCAP,), jnp.int32),
        pltpu.VMEM((NB, G), jnp.int32),
        pltpu.VMEM((G, D), jnp.float32),
        pltpu.VMEM((G, D), jnp.float32),
        pltpu.SemaphoreType.DMA,
        pltpu.SemaphoreType.DMA,
        pltpu.VMEM_SHARED((QH, D), jnp.float32),
    ],
)
def _k2(src_hbm, dst_hbm, y_hbm, s_hbm, *scratch):
    _k2_body(src_hbm, dst_hbm, y_hbm, s_hbm, *scratch)


BR = 640
NBLK = NP // BR


def _k3_kernel(s_ref, y_ref, x_ref, dinv_ref,
               aWc_r, abc_r, aW1_r, ab1_r, aW2_r, ab2_r, aW3_r, ab3_r,
               cWc_r, cbc_r, conc_ref, gsum_ref, csum_ref):
    i = pl.program_id(0)
    dinv = dinv_ref[...]
    z = dinv * (s_ref[...] + y_ref[...])
    xb = x_ref[...]
    hp = jax.lax.Precision.DEFAULT

    ids = lax.broadcasted_iota(jnp.int32, (BR, 1), 0) + i * BR
    valid = ids < N

    h = jnp.maximum(jnp.dot(z, aWc_r[...], precision=hp) + abc_r[...], 0.0)
    h = h + xb
    t = jnp.dot(h, aW1_r[...], precision=hp) + ab1_r[...]
    h = jnp.where(t >= 0, t, 0.01 * t)
    t = jnp.dot(h, aW2_r[...], precision=hp) + ab2_r[...]
    h = jnp.where(t >= 0, t, 0.01 * t)
    logit = jnp.dot(h, aW3_r[...], precision=hp) + ab3_r[...]
    conc = jnp.maximum(logit, 0.0) + jnp.log1p(jnp.exp(-jnp.abs(logit)))
    conc = jnp.where(valid, conc + 1e-20, 0.0)
    conc_ref[...] = conc

    g = jnp.maximum(jnp.dot(z, cWc_r[...], precision=hp) + cbc_r[...], 0.0)
    g = g + xb
    g = jnp.where(valid, g, 0.0)

    @pl.when(i == 0)
    def _():
        gsum_ref[...] = jnp.zeros_like(gsum_ref)
        csum_ref[...] = jnp.zeros_like(csum_ref)

    gsum_ref[...] += jnp.sum(g, axis=0, keepdims=True)
    csum_ref[...] += jnp.sum(conc).reshape(1, 1)


def _k3(s_pad, y_pad, x_pad, dinv2d, aWc, abc, aW1, ab1, aW2, ab2, aW3, ab3,
        cWc, cbc):
    row = lambda i: (i, 0)
    fix = lambda i: (0, 0)
    return pl.pallas_call(
        _k3_kernel,
        grid=(NBLK,),
        in_specs=[
            pl.BlockSpec((BR, D), row),
            pl.BlockSpec((BR, D), row),
            pl.BlockSpec((BR, D), row),
            pl.BlockSpec((BR, 1), row),
            pl.BlockSpec((D, D), fix),
            pl.BlockSpec((1, D), fix),
            pl.BlockSpec((D, D), fix),
            pl.BlockSpec((1, D), fix),
            pl.BlockSpec((D, D), fix),
            pl.BlockSpec((1, D), fix),
            pl.BlockSpec((D, 1), fix),
            pl.BlockSpec((1, 1), fix),
            pl.BlockSpec((D, D), fix),
            pl.BlockSpec((1, D), fix),
        ],
        out_specs=[
            pl.BlockSpec((BR, 1), row),
            pl.BlockSpec((1, D), fix),
            pl.BlockSpec((1, 1), fix),
        ],
        out_shape=[
            jax.ShapeDtypeStruct((NP, 1), jnp.float32),
            jax.ShapeDtypeStruct((1, D), jnp.float32),
            jax.ShapeDtypeStruct((1, 1), jnp.float32),
        ],
    )(s_pad, y_pad, x_pad, dinv2d, aWc, abc, aW1, ab1, aW2, ab2, aW3, ab3,
      cWc, cbc)


def _k4_kernel(conc_ref, csum_ref, gsum_ref,
               cW1_r, cb1_r, cW2_r, cb2_r, cW3_r, cb3_r,
               act_ref, val_ref):
    hp = jax.lax.Precision.DEFAULT
    inv = 1.0 / csum_ref[0, 0]
    act_ref[...] = conc_ref[0:N, :] * inv
    v = jnp.maximum(jnp.dot(gsum_ref[...], cW1_r[...], precision=hp)
                    + cb1_r[...], 0.0)
    v = jnp.maximum(jnp.dot(v, cW2_r[...], precision=hp) + cb2_r[...], 0.0)
    val_ref[...] = jnp.dot(v, cW3_r[...], precision=hp) + cb3_r[...]


def _k4(conc, csum, gsum, cW1, cb1, cW2, cb2, cW3, cb3):
    return pl.pallas_call(
        _k4_kernel,
        out_shape=[
            jax.ShapeDtypeStruct((N, 1), jnp.float32),
            jax.ShapeDtypeStruct((1, 1), jnp.float32),
        ],
    )(conc, csum, gsum, cW1, cb1, cW2, cb2, cW3, cb3)


def kernel(x, edge_index, aWc, abc, aW1, ab1, aW2, ab2, aW3, ab3,
           cWc, cbc, cW1, cb1, cW2, cb2, cW3, cb3):
    src = edge_index[0]
    dst = edge_index[1]
    x_pad = jnp.pad(x, ((0, NP - N), (0, 0)))
    dinv, y_pad = _k1(dst, x_pad)
    s_pad = _k2(src, dst, y_pad)
    conc, gsum, csum = _k3(
        s_pad, y_pad, x_pad, dinv.reshape(NP, 1),
        aWc, abc.reshape(1, D), aW1, ab1.reshape(1, D),
        aW2, ab2.reshape(1, D), aW3, ab3.reshape(1, 1),
        cWc, cbc.reshape(1, D))
    action, value = _k4(conc, csum, gsum,
                        cW1, cb1.reshape(1, D), cW2, cb2.reshape(1, D),
                        cW3, cb3.reshape(1, 1))
    return jnp.concatenate([action.reshape(-1), value.reshape(-1)])


# speedup vs baseline: 1.6325x; 1.6325x over previous
import functools

import jax
import jax.numpy as jnp
from jax import lax
from jax.experimental import pallas as pl
from jax.experimental.pallas import tpu as pltpu
from jax.experimental.pallas import tpu_sc as plsc

N = 10000
D = 256
E = 160000
NC = 2
NS = 16
L = 16
NH = 5120
NP = NC * NH
TR = NH // NS
EC = E // NS
G = 128
C = 2000
PCAP = C + G + 48
NB = PCAP // G
QH = NH // 2
TQ = QH // NS
XB = 64

_MESH = plsc.VectorSubcoreMesh(
    core_axis_name="c", subcore_axis_name="s", num_cores=NC, num_subcores=NS)

_MAGIC = 0x5F3759DF


def _rsqrt_newton(v):
    i = plsc.bitcast(v, jnp.int32)
    i = jnp.int32(_MAGIC) - lax.shift_right_logical(i, 1)
    y = plsc.bitcast(i, jnp.float32)
    for _ in range(3):
        y = y * (1.5 - 0.5 * v * y * y)
    return y


def _k1_body(dst_hbm, x_hbm, dinv_hbm, y_hbm,
             dst_v, hist_v, part_v, red_v, dinv_v, xbuf, hist_sh):
    c = lax.axis_index("c")
    s = lax.axis_index("s")
    lo = c * NH
    lanes = lax.iota(jnp.int32, 16)
    ones = jnp.full((L,), 1.0, jnp.float32)

    def zero_body(j, _):
        for l in range(16):
            hist_v[pl.ds(l * NH + j * L, L)] = jnp.zeros((L,), jnp.float32)
        return 0
    lax.fori_loop(0, NH // L, zero_body, 0)

    pltpu.sync_copy(dst_hbm.at[pl.ds(s * EC, EC)], dst_v)
    lane_base = lanes * NH

    def hist_body(j, _):
        d = dst_v[pl.ds(j * L, L)]
        m = (d >= lo) & (d < lo + NH)
        dloc = jnp.where(m, d - lo, 0)
        plsc.addupdate_scatter(hist_v, [lane_base + dloc], ones, mask=m)
        return 0
    lax.fori_loop(0, EC // L, hist_body, 0)

    def lred_body(j, _):
        acc = hist_v[pl.ds(j * L, L)]
        for l in range(1, 16):
            acc = acc + hist_v[pl.ds(l * NH + j * L, L)]
        part_v[pl.ds(j * L, L)] = acc
        return 0
    lax.fori_loop(0, NH // L, lred_body, 0)

    pltpu.sync_copy(part_v, hist_sh.at[pl.ds(s * NH, NH)])
    plsc.subcore_barrier()
    for t in range(NS):
        pltpu.sync_copy(hist_sh.at[pl.ds(t * NH + s * TR, TR)], red_v.at[t])

    def deg_body(j, _):
        acc = red_v[0, pl.ds(j * L, L)]
        for t in range(1, 16):
            acc = acc + red_v[t, pl.ds(j * L, L)]
        deg = acc + 1.0
        dinv_v[pl.ds(j * L, L)] = _rsqrt_newton(deg)
        return 0
    lax.fori_loop(0, TR // L, deg_body, 0)

    g0 = lo + s * TR
    pltpu.sync_copy(dinv_v, dinv_hbm.at[pl.ds(g0, TR)])

    def y_body(jb, _):
        base = jb * XB
        pltpu.sync_copy(x_hbm.at[pl.ds(g0 + base, XB)], xbuf)
        for q in range(XB // L):
            ch = dinv_v[pl.ds(base + q * L, L)]
            for k in range(L):
                sv = jnp.full((L,), ch[k], jnp.float32)
                r = q * L + k
                for v in range(D // L):
                    xbuf[r, pl.ds(v * L, L)] = xbuf[r, pl.ds(v * L, L)] * sv
        pltpu.sync_copy(xbuf, y_hbm.at[pl.ds(g0 + base, XB)])
        return 0
    lax.fori_loop(0, TR // XB, y_body, 0)


@functools.partial(
    pl.kernel,
    out_type=(jax.ShapeDtypeStruct((NP,), jnp.float32),
              jax.ShapeDtypeStruct((NP, D), jnp.float32)),
    mesh=_MESH,
    compiler_params=pltpu.CompilerParams(needs_layout_passes=False, use_tc_tiling_on_sc=False),
    scratch_types=[
        pltpu.VMEM((EC,), jnp.int32),
        pltpu.VMEM((16 * NH,), jnp.float32),
        pltpu.VMEM((NH,), jnp.float32),
        pltpu.VMEM((NS, TR), jnp.float32),
        pltpu.VMEM((TR,), jnp.float32),
        pltpu.VMEM((XB, D), jnp.float32),
        pltpu.VMEM_SHARED((NS * NH,), jnp.float32),
    ],
)
def _k1(dst_hbm, x_hbm, dinv_hbm, y_hbm, *scratch):
    _k1_body(dst_hbm, x_hbm, dinv_hbm, y_hbm, *scratch)


def _k2_body(src_hbm, dst_hbm, y_hbm, s_hbm,
             src_v, dst_v, psrc, pdst1, pdst2, gbuf0, sem0, acc_sh):
    cc = lax.axis_index("c")
    s = lax.axis_index("s")

    dumm_s = jnp.full((L,), NP - L, jnp.int32)
    dumm_d = jnp.zeros((L,), jnp.int32)

    def gstart(b, buf, sem):
        pltpu.async_copy(y_hbm.at[psrc.at[pl.ds(b * G, G)]], buf, sem)

    def gwait(b, buf, sem):
        pltpu.make_async_copy(
            y_hbm.at[psrc.at[pl.ds(b * G, G)]], buf, sem).wait()

    lo = cc * NH

    def zg_body(j, _):
        for v in range(D // L):
            gbuf0[j, pl.ds(v * L, L)] = jnp.zeros((L,), jnp.float32)
        return 0
    lax.fori_loop(0, G, zg_body, 0)
    pltpu.sync_copy(gbuf0, acc_sh.at[pl.ds(s * TR, G)])
    pltpu.sync_copy(gbuf0, acc_sh.at[pl.ds(s * TR + G, G)])
    pltpu.sync_copy(gbuf0.at[pl.ds(0, TR - 2 * G)],
                    acc_sh.at[pl.ds(s * TR + 2 * G, TR - 2 * G)])
    plsc.subcore_barrier()

    def sub_body(k, _):
        e0 = s * EC + k * C
        pltpu.sync_copy(src_hbm.at[pl.ds(e0, C)], src_v)
        pltpu.sync_copy(dst_hbm.at[pl.ds(e0, C)], dst_v)

        def scan_body(j, cnt):
            d = dst_v[pl.ds(j * L, L)]
            sv = src_v[pl.ds(j * L, L)]
            m = (d >= lo) & (d < lo + NH)
            plsc.store_compressed(psrc.at[pl.ds(cnt, L)], sv, mask=m)
            plsc.store_compressed(pdst1.at[pl.ds(cnt, L)], d - lo, mask=m)
            return cnt + plsc.all_reduce_population_count(m)[0]
        cnt = lax.fori_loop(0, C // L, scan_body, jnp.int32(0))

        for j in range(G // L):
            psrc[pl.ds(cnt + j * L, L)] = dumm_s
            pdst1[pl.ds(cnt + j * L, L)] = dumm_d

        nb = (cnt + (G - 1)) // G

        def p2_body(b, _):
            for t in range(G // L):
                pdst2[b, pl.ds(t * L, L)] = pdst1[pl.ds(b * G + t * L, L)]
            return 0
        lax.fori_loop(0, nb, p2_body, 0)

        def drain_body(b, _):
            pltpu.sync_copy(y_hbm.at[psrc.at[pl.ds(b * G, G)]], gbuf0)
            pltpu.sync_copy(gbuf0, acc_sh.at[pdst2.at[b]], add=True)
            return 0
        lax.fori_loop(0, nb, drain_body, 0)
        return 0
    lax.fori_loop(0, EC // C, sub_body, 0)

    plsc.subcore_barrier()

    g0 = lo + s * TR
    for t in range(2):
        pltpu.sync_copy(acc_sh.at[pl.ds(s * TR + t * G, G)], gbuf0)
        pltpu.sync_copy(gbuf0, s_hbm.at[pl.ds(g0 + t * G, G)])
    pltpu.sync_copy(acc_sh.at[pl.ds(s * TR + 2 * G, TR - 2 * G)],
                    gbuf0.at[pl.ds(0, TR - 2 * G)])
    pltpu.sync_copy(gbuf0.at[pl.ds(0, TR - 2 * G)],
                    s_hbm.at[pl.ds(g0 + 2 * G, TR - 2 * G)])


@functools.partial(
    pl.kernel,
    out_type=jax.ShapeDtypeStruct((NP, D), jnp.float32),
    mesh=_MESH,
    compiler_params=pltpu.CompilerParams(needs_layout_passes=False, use_tc_tiling_on_sc=False),
    scratch_types=[
        pltpu.VMEM((C,), jnp.int32),
        pltpu.VMEM((C,), jnp.int32),
        pltpu.VMEM((PCAP,), jnp.int32),
        pltpu.VMEM((PCAP,), jnp.int32),
        pltpu.VMEM((NB, G), jnp.int32),
        pltpu.VMEM((G, D), jnp.float32),
        pltpu.SemaphoreType.DMA,
        pltpu.VMEM_SHARED((NH, D), jnp.float32),
    ],
)
def _k2(src_hbm, dst_hbm, y_hbm, s_hbm, *scratch):
    _k2_body(src_hbm, dst_hbm, y_hbm, s_hbm, *scratch)


BR = 640
NBLK = NP // BR


def _k3_kernel(s_ref, y_ref, x_ref, dinv_ref,
               aWc_r, abc_r, aW1_r, ab1_r, aW2_r, ab2_r, aW3_r, ab3_r,
               cWc_r, cbc_r, conc_ref, gsum_ref, csum_ref):
    i = pl.program_id(0)
    dinv = dinv_ref[...]
    z = dinv * (s_ref[...] + y_ref[...])
    xb = x_ref[...]
    hp = jax.lax.Precision.DEFAULT

    ids = lax.broadcasted_iota(jnp.int32, (BR, 1), 0) + i * BR
    valid = ids < N

    h = jnp.maximum(jnp.dot(z, aWc_r[...], precision=hp) + abc_r[...], 0.0)
    h = h + xb
    t = jnp.dot(h, aW1_r[...], precision=hp) + ab1_r[...]
    h = jnp.where(t >= 0, t, 0.01 * t)
    t = jnp.dot(h, aW2_r[...], precision=hp) + ab2_r[...]
    h = jnp.where(t >= 0, t, 0.01 * t)
    logit = jnp.dot(h, aW3_r[...], precision=hp) + ab3_r[...]
    conc = jnp.maximum(logit, 0.0) + jnp.log1p(jnp.exp(-jnp.abs(logit)))
    conc = jnp.where(valid, conc + 1e-20, 0.0)
    conc_ref[...] = conc

    g = jnp.maximum(jnp.dot(z, cWc_r[...], precision=hp) + cbc_r[...], 0.0)
    g = g + xb
    g = jnp.where(valid, g, 0.0)

    @pl.when(i == 0)
    def _():
        gsum_ref[...] = jnp.zeros_like(gsum_ref)
        csum_ref[...] = jnp.zeros_like(csum_ref)

    gsum_ref[...] += jnp.sum(g, axis=0, keepdims=True)
    csum_ref[...] += jnp.sum(conc).reshape(1, 1)


def _k3(s_pad, y_pad, x_pad, dinv2d, aWc, abc, aW1, ab1, aW2, ab2, aW3, ab3,
        cWc, cbc):
    row = lambda i: (i, 0)
    fix = lambda i: (0, 0)
    return pl.pallas_call(
        _k3_kernel,
        grid=(NBLK,),
        in_specs=[
            pl.BlockSpec((BR, D), row),
            pl.BlockSpec((BR, D), row),
            pl.BlockSpec((BR, D), row),
            pl.BlockSpec((BR, 1), row),
            pl.BlockSpec((D, D), fix),
            pl.BlockSpec((1, D), fix),
            pl.BlockSpec((D, D), fix),
            pl.BlockSpec((1, D), fix),
            pl.BlockSpec((D, D), fix),
            pl.BlockSpec((1, D), fix),
            pl.BlockSpec((D, 1), fix),
            pl.BlockSpec((1, 1), fix),
            pl.BlockSpec((D, D), fix),
            pl.BlockSpec((1, D), fix),
        ],
        out_specs=[
            pl.BlockSpec((BR, 1), row),
            pl.BlockSpec((1, D), fix),
            pl.BlockSpec((1, 1), fix),
        ],
        out_shape=[
            jax.ShapeDtypeStruct((NP, 1), jnp.float32),
            jax.ShapeDtypeStruct((1, D), jnp.float32),
            jax.ShapeDtypeStruct((1, 1), jnp.float32),
        ],
    )(s_pad, y_pad, x_pad, dinv2d, aWc, abc, aW1, ab1, aW2, ab2, aW3, ab3,
      cWc, cbc)


def _k4_kernel(conc_ref, csum_ref, gsum_ref,
               cW1_r, cb1_r, cW2_r, cb2_r, cW3_r, cb3_r,
               act_ref, val_ref):
    hp = jax.lax.Precision.DEFAULT
    inv = 1.0 / csum_ref[0, 0]
    act_ref[...] = conc_ref[0:N, :] * inv
    v = jnp.maximum(jnp.dot(gsum_ref[...], cW1_r[...], precision=hp)
                    + cb1_r[...], 0.0)
    v = jnp.maximum(jnp.dot(v, cW2_r[...], precision=hp) + cb2_r[...], 0.0)
    val_ref[...] = jnp.dot(v, cW3_r[...], precision=hp) + cb3_r[...]


def _k4(conc, csum, gsum, cW1, cb1, cW2, cb2, cW3, cb3):
    return pl.pallas_call(
        _k4_kernel,
        out_shape=[
            jax.ShapeDtypeStruct((N, 1), jnp.float32),
            jax.ShapeDtypeStruct((1, 1), jnp.float32),
        ],
    )(conc, csum, gsum, cW1, cb1, cW2, cb2, cW3, cb3)


def kernel(x, edge_index, aWc, abc, aW1, ab1, aW2, ab2, aW3, ab3,
           cWc, cbc, cW1, cb1, cW2, cb2, cW3, cb3):
    src = edge_index[0]
    dst = edge_index[1]
    x_pad = jnp.pad(x, ((0, NP - N), (0, 0)))
    dinv, y_pad = _k1(dst, x_pad)
    s_pad = _k2(src, dst, y_pad)
    conc, gsum, csum = _k3(
        s_pad, y_pad, x_pad, dinv.reshape(NP, 1),
        aWc, abc.reshape(1, D), aW1, ab1.reshape(1, D),
        aW2, ab2.reshape(1, D), aW3, ab3.reshape(1, 1),
        cWc, cbc.reshape(1, D))
    action, value = _k4(conc, csum, gsum,
                        cW1, cb1.reshape(1, D), cW2, cb2.reshape(1, D),
                        cW3, cb3.reshape(1, 1))
    return jnp.concatenate([action.reshape(-1), value.reshape(-1)])
